# baseline (speedup 1.0000x reference)
import functools

import jax
import jax.numpy as jnp
from jax import lax
from jax.experimental import pallas as pl
from jax.experimental.pallas import tpu as pltpu
from jax.experimental.pallas import tpu_sc as plsc

N = 10000
E = 160000
D_IN = 256
D_HID = 128
D_OUT = 128

NC = 2
NS = 16
NW = NC * NS
CHUNK = 128
NCHUNK = 40
E_PAD = NW * NCHUNK * CHUNK
N_PAD = 10240
RPT = N_PAD // NS

def _deg_body(dst_hbm, ones_hbm, zeros_hbm, out_hbm, didx_v, ones_v, acc_sh,
              sem):
  cid = lax.axis_index("c")
  sid = lax.axis_index("s")
  wid = cid * NS + sid
  pltpu.sync_copy(zeros_hbm, acc_sh.at[pl.ds(sid * RPT, RPT)])
  pltpu.sync_copy(dst_hbm.at[wid], didx_v)
  pltpu.sync_copy(ones_hbm, ones_v)
  plsc.subcore_barrier()

  def body(j, carry):
    pltpu.sync_copy(ones_v, acc_sh.at[didx_v.at[j]], add=True)
    return carry

  lax.fori_loop(0, NCHUNK, body, 0)
  plsc.subcore_barrier()
  pltpu.sync_copy(acc_sh.at[pl.ds(sid * RPT, RPT)],
                  out_hbm.at[pl.ds(cid * N_PAD + sid * RPT, RPT)])


@functools.cache
def _deg_sc():
  mesh = plsc.VectorSubcoreMesh(
      core_axis_name="c", subcore_axis_name="s", num_cores=NC,
      num_subcores=NS)
  return pl.kernel(
      _deg_body,
      out_type=jax.ShapeDtypeStruct((NC * N_PAD, 16), jnp.float32),
      mesh=mesh,
      scratch_types=[
          pltpu.VMEM((NCHUNK, CHUNK), jnp.int32),
          pltpu.VMEM((CHUNK, 16), jnp.float32),
          pltpu.VMEM_SHARED((N_PAD, 16), jnp.float32),
          pltpu.SemaphoreType.DMA,
      ],
  )


NBUF = 2


def _scatter_body(g_hbm, src_hbm, dst_hbm, zeros_hbm, out_hbm, sidx_v, didx_v,
                  rows_v, acc_sh, s0, s1):
  sems = (s0, s1)
  cid = lax.axis_index("c")
  sid = lax.axis_index("s")
  wid = cid * NS + sid
  pltpu.sync_copy(zeros_hbm, acc_sh.at[pl.ds(sid * RPT, RPT)])
  pltpu.sync_copy(src_hbm.at[wid], sidx_v)
  pltpu.sync_copy(dst_hbm.at[wid], didx_v)

  for b in range(NBUF):
    pltpu.async_copy(g_hbm.at[sidx_v.at[b]], rows_v.at[b], sems[b])
  plsc.subcore_barrier()

  @pl.loop(0, NCHUNK, step=NBUF)
  def _(j):
    for b in range(NBUF):
      pltpu.make_async_copy(g_hbm.at[sidx_v.at[0]], rows_v.at[b],
                            sems[b]).wait()
      pltpu.sync_copy(rows_v.at[b], acc_sh.at[didx_v.at[j + b]], add=True)
      pltpu.async_copy(g_hbm.at[sidx_v.at[j + b + NBUF]], rows_v.at[b],
                       sems[b])

  for b in range(NBUF):
    pltpu.make_async_copy(g_hbm.at[sidx_v.at[0]], rows_v.at[b],
                          sems[b]).wait()
  plsc.subcore_barrier()
  pltpu.sync_copy(acc_sh.at[pl.ds(sid * RPT, RPT)],
                  out_hbm.at[pl.ds(cid * N_PAD + sid * RPT, RPT)])


@functools.cache
def _scatter_sc():
  mesh = plsc.VectorSubcoreMesh(
      core_axis_name="c", subcore_axis_name="s", num_cores=NC,
      num_subcores=NS)
  return pl.kernel(
      _scatter_body,
      out_type=jax.ShapeDtypeStruct((NC * N_PAD, D_HID), jnp.float32),
      mesh=mesh,
      scratch_types=[
          pltpu.VMEM((NCHUNK + NBUF, CHUNK), jnp.int32),
          pltpu.VMEM((NCHUNK, CHUNK), jnp.int32),
          pltpu.VMEM((NBUF, CHUNK, D_HID), jnp.float32),
          pltpu.VMEM_SHARED((N_PAD, D_HID), jnp.float32),
          pltpu.SemaphoreType.DMA,
          pltpu.SemaphoreType.DMA,
      ],
  )


_BR = 400


def _dis_from(deg_ref):
  deg = deg_ref[0] + deg_ref[1] + 1.0
  return lax.rsqrt(deg[:, :1])


def _mm1_body(x_ref, w_ref, deg_ref, out_ref):
  dis = _dis_from(deg_ref)
  hw = jnp.dot(x_ref[...], w_ref[...], preferred_element_type=jnp.float32)
  out_ref[...] = hw * dis


def _mm2_body(acc_ref, g_ref, deg_ref, b_ref, w_ref, out_ref):
  dis = _dis_from(deg_ref)
  h = dis * (acc_ref[0] + acc_ref[1] + g_ref[...]) + b_ref[...]
  h = jnp.maximum(h, 0.0)
  hw = jnp.dot(h, w_ref[...], preferred_element_type=jnp.float32)
  out_ref[...] = hw * dis


def _fin_body(acc_ref, g_ref, deg_ref, b_ref, out_ref):
  dis = _dis_from(deg_ref)
  out_ref[...] = dis * (acc_ref[0] + acc_ref[1] + g_ref[...]) + b_ref[...]


def _row_specs(d_feat):
  acc = pl.BlockSpec((2, _BR, d_feat), lambda i: (0, i, 0))
  g = pl.BlockSpec((_BR, d_feat), lambda i: (i, 0))
  deg = pl.BlockSpec((2, _BR, 16), lambda i: (0, i, 0))
  return acc, g, deg


def _mm1(x, w1, deg2):
  _, gspec, degspec = _row_specs(D_IN)
  return pl.pallas_call(
      _mm1_body,
      grid=(N // _BR,),
      in_specs=[
          pl.BlockSpec((_BR, D_IN), lambda i: (i, 0)),
          pl.BlockSpec((D_IN, D_HID), lambda i: (0, 0)),
          degspec,
      ],
      out_specs=pl.BlockSpec((_BR, D_HID), lambda i: (i, 0)),
      out_shape=jax.ShapeDtypeStruct((N, D_HID), jnp.float32),
  )(x, w1, deg2)


def _mm2(acc2, g, deg2, b1, w2):
  accspec, gspec, degspec = _row_specs(D_HID)
  return pl.pallas_call(
      _mm2_body,
      grid=(N // _BR,),
      in_specs=[
          accspec,
          gspec,
          degspec,
          pl.BlockSpec((1, D_HID), lambda i: (0, 0)),
          pl.BlockSpec((D_HID, D_OUT), lambda i: (0, 0)),
      ],
      out_specs=pl.BlockSpec((_BR, D_OUT), lambda i: (i, 0)),
      out_shape=jax.ShapeDtypeStruct((N, D_OUT), jnp.float32),
  )(acc2, g, deg2, b1, w2)


def _fin(acc2, g, deg2, b2):
  accspec, gspec, degspec = _row_specs(D_OUT)
  return pl.pallas_call(
      _fin_body,
      grid=(N // _BR,),
      in_specs=[
          accspec,
          gspec,
          degspec,
          pl.BlockSpec((1, D_OUT), lambda i: (0, 0)),
      ],
      out_specs=pl.BlockSpec((_BR, D_OUT), lambda i: (i, 0)),
      out_shape=jax.ShapeDtypeStruct((N, D_OUT), jnp.float32),
  )(acc2, g, deg2, b2)


@jax.jit
def _run(x, edge_index, w1, b1, w2, b2):
  ei = edge_index.astype(jnp.int32)
  pad = E_PAD - E
  src_p = jnp.concatenate([ei[0], jnp.zeros((pad,), jnp.int32)])
  dst_p = jnp.concatenate([ei[1], jnp.full((pad,), N, jnp.int32)])
  src_p = src_p.reshape(NW, NCHUNK, CHUNK)
  dst_p = dst_p.reshape(NW, NCHUNK, CHUNK)
  src_p = jnp.concatenate(
      [src_p, jnp.zeros((NW, NBUF, CHUNK), jnp.int32)], axis=1)

  ones16 = jnp.ones((CHUNK, 16), jnp.float32)
  zeros16 = jnp.zeros((RPT, 16), jnp.float32)
  zeros128 = jnp.zeros((RPT, D_HID), jnp.float32)

  deg2 = _deg_sc()(dst_p, ones16, zeros16).reshape(NC, N_PAD, 16)

  g1 = _mm1(x, w1, deg2)
  acc1 = _scatter_sc()(g1, src_p, dst_p, zeros128).reshape(NC, N_PAD, D_HID)
  g2 = _mm2(acc1, g1, deg2, b1.reshape(1, D_HID), w2)
  acc2 = _scatter_sc()(g2, src_p, dst_p, zeros128).reshape(NC, N_PAD, D_OUT)
  return _fin(acc2, g2, deg2, b2.reshape(1, D_OUT))


def kernel(x, edge_index, cache_name, W1, b1, W2, b2):
  return _run(x, edge_index, W1, b1, W2, b2)


# speedup vs baseline: 1.7566x; 1.7566x over previous
import functools

import jax
import jax.numpy as jnp
from jax import lax
from jax.experimental import pallas as pl
from jax.experimental.pallas import tpu as pltpu
from jax.experimental.pallas import tpu_sc as plsc

N = 10000
E = 160000
D_IN = 256
D_HID = 128
D_OUT = 128

NC = 2
NS = 16
NW = NC * NS
CHUNK = 128
NCHUNK = 40
E_PAD = NW * NCHUNK * CHUNK
N_PAD = 10240
RPT = N_PAD // NS

def _deg_body(dst_hbm, ones_hbm, zeros_hbm, out_hbm, didx_v, ones_v, acc_sh,
              sem):
  cid = lax.axis_index("c")
  sid = lax.axis_index("s")
  wid = cid * NS + sid
  pltpu.sync_copy(zeros_hbm, acc_sh.at[pl.ds(sid * RPT, RPT)])
  pltpu.sync_copy(dst_hbm.at[wid], didx_v)
  pltpu.sync_copy(ones_hbm, ones_v)
  plsc.subcore_barrier()

  def body(j, carry):
    pltpu.sync_copy(ones_v, acc_sh.at[didx_v.at[j]], add=True)
    return carry

  lax.fori_loop(0, NCHUNK, body, 0)
  plsc.subcore_barrier()
  pltpu.sync_copy(acc_sh.at[pl.ds(sid * RPT, RPT)],
                  out_hbm.at[pl.ds(cid * N_PAD + sid * RPT, RPT)])


@functools.cache
def _deg_sc():
  mesh = plsc.VectorSubcoreMesh(
      core_axis_name="c", subcore_axis_name="s", num_cores=NC,
      num_subcores=NS)
  return pl.kernel(
      _deg_body,
      out_type=jax.ShapeDtypeStruct((NC * N_PAD, 16), jnp.float32),
      mesh=mesh,
      scratch_types=[
          pltpu.VMEM((NCHUNK, CHUNK), jnp.int32),
          pltpu.VMEM((CHUNK, 16), jnp.float32),
          pltpu.VMEM_SHARED((N_PAD, 16), jnp.float32),
          pltpu.SemaphoreType.DMA,
      ],
  )


NBUF = 2


def _scatter_body(g_hbm, src_hbm, dst_hbm, zeros_hbm, out_hbm, sidx_v, didx_v,
                  rows_v, acc_sh, s0, s1):
  sems = (s0, s1)
  cid = lax.axis_index("c")
  sid = lax.axis_index("s")
  wid = cid * NS + sid
  pltpu.sync_copy(zeros_hbm, acc_sh.at[pl.ds(sid * RPT, RPT)])
  pltpu.sync_copy(src_hbm.at[wid], sidx_v)
  pltpu.sync_copy(dst_hbm.at[wid], didx_v)

  plsc.subcore_barrier()

  @pl.loop(0, NCHUNK, step=NBUF)
  def _(j):
    cps = [
        pltpu.async_copy(g_hbm.at[sidx_v.at[j + b]], rows_v.at[b], sems[b])
        for b in range(NBUF)
    ]
    for b in range(NBUF):
      cps[b].wait()
      pltpu.sync_copy(rows_v.at[b], acc_sh.at[didx_v.at[j + b]], add=True)

  plsc.subcore_barrier()
  pltpu.sync_copy(acc_sh.at[pl.ds(sid * RPT, RPT)],
                  out_hbm.at[pl.ds(cid * N_PAD + sid * RPT, RPT)])


@functools.cache
def _scatter_sc():
  mesh = plsc.VectorSubcoreMesh(
      core_axis_name="c", subcore_axis_name="s", num_cores=NC,
      num_subcores=NS)
  return pl.kernel(
      _scatter_body,
      out_type=jax.ShapeDtypeStruct((NC * N_PAD, D_HID), jnp.float32),
      mesh=mesh,
      scratch_types=[
          pltpu.VMEM((NCHUNK, CHUNK), jnp.int32),
          pltpu.VMEM((NCHUNK, CHUNK), jnp.int32),
          pltpu.VMEM((NBUF, CHUNK, D_HID), jnp.float32),
          pltpu.VMEM_SHARED((N_PAD, D_HID), jnp.float32),
          pltpu.SemaphoreType.DMA,
          pltpu.SemaphoreType.DMA,
      ],
  )


_BR = 400


def _dis_from(deg_ref):
  deg = deg_ref[0] + deg_ref[1] + 1.0
  return lax.rsqrt(deg[:, :1])


def _mm1_body(x_ref, w_ref, deg_ref, out_ref):
  dis = _dis_from(deg_ref)
  hw = jnp.dot(x_ref[...], w_ref[...], preferred_element_type=jnp.float32)
  out_ref[...] = hw * dis


def _mm2_body(acc_ref, g_ref, deg_ref, b_ref, w_ref, out_ref):
  dis = _dis_from(deg_ref)
  h = dis * (acc_ref[0] + acc_ref[1] + g_ref[...]) + b_ref[...]
  h = jnp.maximum(h, 0.0)
  hw = jnp.dot(h, w_ref[...], preferred_element_type=jnp.float32)
  out_ref[...] = hw * dis


def _fin_body(acc_ref, g_ref, deg_ref, b_ref, out_ref):
  dis = _dis_from(deg_ref)
  out_ref[...] = dis * (acc_ref[0] + acc_ref[1] + g_ref[...]) + b_ref[...]


def _row_specs(d_feat):
  acc = pl.BlockSpec((2, _BR, d_feat), lambda i: (0, i, 0))
  g = pl.BlockSpec((_BR, d_feat), lambda i: (i, 0))
  deg = pl.BlockSpec((2, _BR, 16), lambda i: (0, i, 0))
  return acc, g, deg


def _mm1(x, w1, deg2):
  _, gspec, degspec = _row_specs(D_IN)
  return pl.pallas_call(
      _mm1_body,
      grid=(N // _BR,),
      in_specs=[
          pl.BlockSpec((_BR, D_IN), lambda i: (i, 0)),
          pl.BlockSpec((D_IN, D_HID), lambda i: (0, 0)),
          degspec,
      ],
      out_specs=pl.BlockSpec((_BR, D_HID), lambda i: (i, 0)),
      out_shape=jax.ShapeDtypeStruct((N, D_HID), jnp.float32),
  )(x, w1, deg2)


def _mm2(acc2, g, deg2, b1, w2):
  accspec, gspec, degspec = _row_specs(D_HID)
  return pl.pallas_call(
      _mm2_body,
      grid=(N // _BR,),
      in_specs=[
          accspec,
          gspec,
          degspec,
          pl.BlockSpec((1, D_HID), lambda i: (0, 0)),
          pl.BlockSpec((D_HID, D_OUT), lambda i: (0, 0)),
      ],
      out_specs=pl.BlockSpec((_BR, D_OUT), lambda i: (i, 0)),
      out_shape=jax.ShapeDtypeStruct((N, D_OUT), jnp.float32),
  )(acc2, g, deg2, b1, w2)


def _fin(acc2, g, deg2, b2):
  accspec, gspec, degspec = _row_specs(D_OUT)
  return pl.pallas_call(
      _fin_body,
      grid=(N // _BR,),
      in_specs=[
          accspec,
          gspec,
          degspec,
          pl.BlockSpec((1, D_OUT), lambda i: (0, 0)),
      ],
      out_specs=pl.BlockSpec((_BR, D_OUT), lambda i: (i, 0)),
      out_shape=jax.ShapeDtypeStruct((N, D_OUT), jnp.float32),
  )(acc2, g, deg2, b2)


@jax.jit
def _run(x, edge_index, w1, b1, w2, b2):
  ei = edge_index.astype(jnp.int32)
  pad = E_PAD - E
  src_p = jnp.concatenate([ei[0], jnp.zeros((pad,), jnp.int32)])
  dst_p = jnp.concatenate([ei[1], jnp.full((pad,), N, jnp.int32)])
  src_p = src_p.reshape(NW, NCHUNK, CHUNK)
  dst_p = dst_p.reshape(NW, NCHUNK, CHUNK)

  ones16 = jnp.ones((CHUNK, 16), jnp.float32)
  zeros16 = jnp.zeros((RPT, 16), jnp.float32)
  zeros128 = jnp.zeros((RPT, D_HID), jnp.float32)

  deg2 = _deg_sc()(dst_p, ones16, zeros16).reshape(NC, N_PAD, 16)

  g1 = _mm1(x, w1, deg2)
  acc1 = _scatter_sc()(g1, src_p, dst_p, zeros128).reshape(NC, N_PAD, D_HID)
  g2 = _mm2(acc1, g1, deg2, b1.reshape(1, D_HID), w2)
  acc2 = _scatter_sc()(g2, src_p, dst_p, zeros128).reshape(NC, N_PAD, D_OUT)
  return _fin(acc2, g2, deg2, b2.reshape(1, D_OUT))


def kernel(x, edge_index, cache_name, W1, b1, W2, b2):
  return _run(x, edge_index, W1, b1, W2, b2)
